# Initial kernel scaffold; baseline (speedup 1.0000x reference)
import jax
import jax.numpy as jnp
from jax.experimental import pallas as pl


def kernel(x, edge_index, W_l, b_l, W_r, gamma, beta):
    raise NotImplementedError("write your pallas kernel here")


# speedup vs baseline: 8.8622x; 8.8622x over previous
import functools

import jax
import jax.numpy as jnp
from jax import lax
from jax.experimental import pallas as pl
from jax.experimental.pallas import tpu as pltpu
from jax.experimental.pallas import tpu_sc as plsc

N = 10000
D = 128
E = 320000
EPS = 1e-5

NC = 2
NS = 16
CH = 128
CPW = 80
E_PAD = NC * NS * CPW * CH
NPAD = 112
NA = N + NPAD
RPT = NA // NS

_SC_MESH = dict(
    mesh=plsc.VectorSubcoreMesh(core_axis_name="c", subcore_axis_name="s",
                                num_cores=NC, num_subcores=NS),
)


def _unpack_chunk(pk_v, j, src_row, dst_row):
    for k in range(CH // 16):
        v = pk_v[j, pl.ds(k * 16, 16)]
        if src_row is not None:
            src_row[pl.ds(k * 16, 16)] = lax.bitwise_and(v, jnp.int32(0xFFFF))
        dst_row[pl.ds(k * 16, 16)] = lax.shift_right_logical(v, jnp.int32(16))


def _sc_agg_body(x_hbm, pk_hbm, zagg_hbm, agg_out,
                 pk_v, src_row, dst_row, rows_v, agg_sh, gsem):
    c = lax.axis_index("c")
    s = lax.axis_index("s")
    r0 = s * RPT

    pltpu.sync_copy(zagg_hbm.at[pl.ds(r0, RPT)], agg_sh.at[pl.ds(r0, RPT)])
    pltpu.sync_copy(pk_hbm.at[c, s], pk_v)

    plsc.subcore_barrier()

    def _chunk(j, carry):
        _unpack_chunk(pk_v, j, src_row, dst_row)
        pltpu.async_copy(x_hbm.at[src_row], rows_v, gsem).wait()
        pltpu.sync_copy(rows_v, agg_sh.at[dst_row], add=True)
        return carry
    lax.fori_loop(0, CPW, _chunk, 0)

    plsc.subcore_barrier()
    pltpu.sync_copy(agg_sh.at[pl.ds(r0, RPT)], agg_out.at[c, pl.ds(r0, RPT)])


_sc_aggregate = functools.partial(
    pl.kernel,
    out_type=jax.ShapeDtypeStruct((NC, NA, D), jnp.float32),
    scratch_types=[
        pltpu.VMEM((CPW, CH), jnp.int32),
        pltpu.VMEM((CH,), jnp.int32),
        pltpu.VMEM((CH,), jnp.int32),
        pltpu.VMEM((CH, D), jnp.float32),
        pltpu.VMEM_SHARED((NA, D), jnp.float32),
        pltpu.SemaphoreType.DMA,
    ],
    **_SC_MESH,
)(_sc_agg_body)


def _sc_cnt_body(pk_hbm, zcnt_hbm, cnt_out,
                 pk_v, dst_row, ones_v, cnt_sh):
    c = lax.axis_index("c")
    s = lax.axis_index("s")
    r0 = s * RPT

    pltpu.sync_copy(zcnt_hbm.at[pl.ds(r0, RPT)], cnt_sh.at[pl.ds(r0, RPT)])

    def _fill(i, carry):
        ones_v[i, :] = jnp.ones((16,), jnp.float32)
        return carry
    lax.fori_loop(0, CH, _fill, 0)

    pltpu.sync_copy(pk_hbm.at[c, s], pk_v)

    plsc.subcore_barrier()

    def _chunk(j, carry):
        _unpack_chunk(pk_v, j, None, dst_row)
        pltpu.sync_copy(ones_v, cnt_sh.at[dst_row], add=True)
        return carry
    lax.fori_loop(0, CPW, _chunk, 0)

    plsc.subcore_barrier()
    pltpu.sync_copy(cnt_sh.at[pl.ds(r0, RPT)], cnt_out.at[c, pl.ds(r0, RPT)])


_sc_count = functools.partial(
    pl.kernel,
    out_type=jax.ShapeDtypeStruct((NC, NA, 16), jnp.float32),
    scratch_types=[
        pltpu.VMEM((CPW, CH), jnp.int32),
        pltpu.VMEM((CH,), jnp.int32),
        pltpu.VMEM((CH, 16), jnp.float32),
        pltpu.VMEM_SHARED((NA, 16), jnp.float32),
    ],
    **_SC_MESH,
)(_sc_cnt_body)


def _tc_body(agg_ref, cnt_ref, x_ref, wl_ref, bl_ref, wr_ref, g_ref, b_ref,
             out_ref):
    agg = agg_ref[0, :N, :] + agg_ref[1, :N, :]
    cnt = cnt_ref[0, :N, :1] + cnt_ref[1, :N, :1]
    mean = agg / jnp.maximum(cnt, 1.0)
    x = x_ref[...]
    h = (lax.dot_general(mean, wl_ref[...], (((1,), (1,)), ((), ())),
                         preferred_element_type=jnp.float32)
         + lax.dot_general(x, wr_ref[...], (((1,), (1,)), ((), ())),
                           preferred_element_type=jnp.float32)
         + bl_ref[...])
    mu = jnp.mean(h, axis=0, keepdims=True)
    var = jnp.mean(h * h, axis=0, keepdims=True) - mu * mu
    y = g_ref[...] * (h - mu) * lax.rsqrt(var + EPS) + b_ref[...]
    out_ref[...] = jnp.maximum(y, 0.0)


def kernel(x, edge_index, W_l, b_l, W_r, gamma, beta):
    src = edge_index[0].astype(jnp.int32)
    dst = edge_index[1].astype(jnp.int32)
    npad = E_PAD - E
    pad_ids = lax.iota(jnp.int32, npad)
    src_p = jnp.concatenate([src, pad_ids % N])
    dst_p = jnp.concatenate([dst, N + pad_ids % NPAD])
    pk_r = (jnp.left_shift(dst_p, 16) | src_p).reshape(NC, NS, CPW, CH)
    zagg = jnp.zeros((NA, D), jnp.float32)
    zcnt = jnp.zeros((NA, 16), jnp.float32)

    agg_part = _sc_aggregate(x, pk_r, zagg)
    cnt_part = _sc_count(pk_r, zcnt)

    return pl.pallas_call(
        _tc_body,
        out_shape=jax.ShapeDtypeStruct((N, D), jnp.float32),
    )(agg_part, cnt_part, x, W_l, b_l.reshape(1, D), W_r,
      gamma.reshape(1, D), beta.reshape(1, D))


# speedup vs baseline: 12.2709x; 1.3846x over previous
import functools

import jax
import jax.numpy as jnp
from jax import lax
from jax.experimental import pallas as pl
from jax.experimental.pallas import tpu as pltpu
from jax.experimental.pallas import tpu_sc as plsc

N = 10000
D = 128
E = 320000
EPS = 1e-5

NC = 2
NS = 16
CH = 128
CPW = 80
E_PAD = NC * NS * CPW * CH
NPAD = 112
NA = N + NPAD
RPT = NA // NS

_SC_MESH = dict(
    mesh=plsc.VectorSubcoreMesh(core_axis_name="c", subcore_axis_name="s",
                                num_cores=NC, num_subcores=NS),
)


def _unpack_chunk(pk_v, j, src_row, dst_row):
    for k in range(CH // 16):
        v = pk_v[j, pl.ds(k * 16, 16)]
        if src_row is not None:
            src_row[pl.ds(k * 16, 16)] = lax.bitwise_and(v, jnp.int32(0xFFFF))
        dst_row[pl.ds(k * 16, 16)] = lax.shift_right_logical(v, jnp.int32(16))


NBUF = 2


def _sc_agg_body(x_hbm, pk_hbm, zagg_hbm, agg_out,
                 pk_v, src_rows, dst_rows, rows_bufs, agg_sh, gsems):
    c = lax.axis_index("c")
    s = lax.axis_index("s")
    r0 = s * RPT

    pltpu.sync_copy(zagg_hbm.at[pl.ds(r0, RPT)], agg_sh.at[pl.ds(r0, RPT)])
    pltpu.sync_copy(pk_hbm.at[c, s], pk_v)

    plsc.subcore_barrier()

    for b in range(NBUF):
        _unpack_chunk(pk_v, b, src_rows[b], dst_rows[b])
        pltpu.async_copy(x_hbm.at[src_rows[b]], rows_bufs[b], gsems[b])

    def _round(jj, carry):
        j0 = jj * NBUF
        for b in range(NBUF):
            pltpu.make_async_copy(x_hbm.at[src_rows[b]], rows_bufs[b],
                                  gsems[b]).wait()
            pltpu.sync_copy(rows_bufs[b], agg_sh.at[dst_rows[b]], add=True)
            nxt = j0 + b + NBUF

            @pl.when(nxt < CPW)
            def _():
                _unpack_chunk(pk_v, nxt, src_rows[b], dst_rows[b])
                pltpu.async_copy(x_hbm.at[src_rows[b]], rows_bufs[b],
                                 gsems[b])
        return carry
    lax.fori_loop(0, CPW // NBUF, _round, 0)

    plsc.subcore_barrier()
    pltpu.sync_copy(agg_sh.at[pl.ds(r0, RPT)], agg_out.at[c, pl.ds(r0, RPT)])


_sc_aggregate = functools.partial(
    pl.kernel,
    out_type=jax.ShapeDtypeStruct((NC, NA, D), jnp.float32),
    scratch_types=[
        pltpu.VMEM((CPW, CH), jnp.int32),
        [pltpu.VMEM((CH,), jnp.int32)] * NBUF,
        [pltpu.VMEM((CH,), jnp.int32)] * NBUF,
        [pltpu.VMEM((CH, D), jnp.float32)] * NBUF,
        pltpu.VMEM_SHARED((NA, D), jnp.float32),
        [pltpu.SemaphoreType.DMA] * NBUF,
    ],
    **_SC_MESH,
)(_sc_agg_body)


def _sc_cnt_body(pk_hbm, zcnt_hbm, cnt_out,
                 pk_v, dst_row, ones_v, cnt_sh):
    c = lax.axis_index("c")
    s = lax.axis_index("s")
    r0 = s * RPT

    pltpu.sync_copy(zcnt_hbm.at[pl.ds(r0, RPT)], cnt_sh.at[pl.ds(r0, RPT)])

    def _fill(i, carry):
        ones_v[i, :] = jnp.ones((16,), jnp.float32)
        return carry
    lax.fori_loop(0, CH, _fill, 0)

    pltpu.sync_copy(pk_hbm.at[c, s], pk_v)

    plsc.subcore_barrier()

    def _chunk(j, carry):
        _unpack_chunk(pk_v, j, None, dst_row)
        pltpu.sync_copy(ones_v, cnt_sh.at[dst_row], add=True)
        return carry
    lax.fori_loop(0, CPW, _chunk, 0)

    plsc.subcore_barrier()
    pltpu.sync_copy(cnt_sh.at[pl.ds(r0, RPT)], cnt_out.at[c, pl.ds(r0, RPT)])


_sc_count = functools.partial(
    pl.kernel,
    out_type=jax.ShapeDtypeStruct((NC, NA, 16), jnp.float32),
    scratch_types=[
        pltpu.VMEM((CPW, CH), jnp.int32),
        pltpu.VMEM((CH,), jnp.int32),
        pltpu.VMEM((CH, 16), jnp.float32),
        pltpu.VMEM_SHARED((NA, 16), jnp.float32),
    ],
    **_SC_MESH,
)(_sc_cnt_body)


def _tc_body(agg_ref, cnt_ref, x_ref, wl_ref, bl_ref, wr_ref, g_ref, b_ref,
             out_ref):
    agg = agg_ref[0, :N, :] + agg_ref[1, :N, :]
    cnt = cnt_ref[0, :N, :1] + cnt_ref[1, :N, :1]
    mean = agg / jnp.maximum(cnt, 1.0)
    x = x_ref[...]
    h = (lax.dot_general(mean, wl_ref[...], (((1,), (1,)), ((), ())),
                         preferred_element_type=jnp.float32)
         + lax.dot_general(x, wr_ref[...], (((1,), (1,)), ((), ())),
                           preferred_element_type=jnp.float32)
         + bl_ref[...])
    mu = jnp.mean(h, axis=0, keepdims=True)
    var = jnp.mean(h * h, axis=0, keepdims=True) - mu * mu
    y = g_ref[...] * (h - mu) * lax.rsqrt(var + EPS) + b_ref[...]
    out_ref[...] = jnp.maximum(y, 0.0)


def kernel(x, edge_index, W_l, b_l, W_r, gamma, beta):
    src = edge_index[0].astype(jnp.int32)
    dst = edge_index[1].astype(jnp.int32)
    npad = E_PAD - E
    pad_ids = lax.iota(jnp.int32, npad)
    src_p = jnp.concatenate([src, pad_ids % N])
    dst_p = jnp.concatenate([dst, N + pad_ids % NPAD])
    pk_r = (jnp.left_shift(dst_p, 16) | src_p).reshape(NC, NS, CPW, CH)
    zagg = jnp.zeros((NA, D), jnp.float32)
    zcnt = jnp.zeros((NA, 16), jnp.float32)

    agg_part = _sc_aggregate(x, pk_r, zagg)
    cnt_part = _sc_count(pk_r, zcnt)

    return pl.pallas_call(
        _tc_body,
        out_shape=jax.ShapeDtypeStruct((N, D), jnp.float32),
    )(agg_part, cnt_part, x, W_l, b_l.reshape(1, D), W_r,
      gamma.reshape(1, D), beta.reshape(1, D))
